# baseline (speedup 1.0000x reference)
import jax
import jax.numpy as jnp
from jax import lax
from jax.experimental import pallas as pl
from jax.experimental.pallas import tpu as pltpu
from jax.experimental.pallas import tpu_sc as plsc

B, L, D, BIN = 4096, 200, 64, 12
BIN_ALPHA = 1.0
PAD_TOKEN_ID = 0.0

_NW = 32
_RPW = B // _NW
_CB = 2
_E = _CB * L
_NCH = _RPW // _CB
_LN = 16

_W1, _B1, _B2, _W2, _EMB, _PAD = 0, 16, 32, 48, 240, 1008
_WT = 1072


def _splat_i32(v):
    return jnp.zeros((_LN,), jnp.int32) + v


def _bcast(vec, lane):
    return vec.at[_splat_i32(lane)].get(mode="promise_in_bounds")


def _sc_body(wtf, out_hbm, xv, wtv, wv, ov):
    wid = lax.axis_index("s") * 2 + lax.axis_index("c")
    pltpu.sync_copy(wtf.at[pl.ds(0, _WT)], wtv)
    w1v = wtv[pl.ds(_W1, _LN)]
    b1v = wtv[pl.ds(_B1, _LN)]
    b2v = wtv[pl.ds(_B2, _LN)]
    w1s = [_bcast(w1v, k) for k in range(BIN)]
    b1s = [_bcast(b1v, k) for k in range(BIN)]
    b2s = [_bcast(b2v, k) for k in range(BIN)]
    w2rows = [wtv[pl.ds(_W2 + 16 * j, _LN)] for j in range(BIN)]

    def chunk_body(c, _):
        base = (wid * _RPW + c * _CB) * L
        pltpu.sync_copy(wtf.at[pl.ds(_WT + base, _E)], xv)

        def group_body(g, _):
            xg = xv[pl.ds(g * _LN, _LN)]
            h = [xg * w1s[k] + b1s[k] for k in range(BIN)]
            h = [jnp.maximum(a, 0.1 * a) for a in h]
            logits = []
            for k in range(BIN):
                acc = BIN_ALPHA * h[k] + b2s[k]
                for j in range(BIN):
                    acc = acc + h[j] * _bcast(w2rows[j], k)
                logits.append(acc)
            m = logits[0]
            for k in range(1, BIN):
                m = jnp.maximum(m, logits[k])
            e = [jnp.exp(v - m) for v in logits]
            s = e[0]
            for k in range(1, BIN):
                s = s + e[k]
            r = 1.0 / s
            pk = xg == PAD_TOKEN_ID
            w13 = [jnp.where(pk, 0.0, v * r) for v in e]
            w13.append(jnp.where(pk, 1.0, 0.0))
            for k in range(BIN + 1):
                wv[pl.ds(k * _E + g * _LN, _LN)] = w13[k]
            return 0

        lax.fori_loop(0, _E // _LN, group_body, 0)

        embvs = [[wtv[pl.ds(_EMB + k * D + i * _LN, _LN)] for i in range(4)]
                 for k in range(BIN)]
        padvs = [wtv[pl.ds(_PAD + i * _LN, _LN)] for i in range(4)]

        def elem_body(n, _):
            seg = (n // _LN) * _LN
            lane = n - seg
            ws = [_bcast(wv[pl.ds(k * _E + seg, _LN)], lane)
                  for k in range(BIN + 1)]
            for i in range(4):
                acc = ws[BIN] * padvs[i]
                for k in range(BIN):
                    acc = acc + ws[k] * embvs[k][i]
                ov[pl.ds(n * D + i * _LN, _LN)] = acc
            return 0

        lax.fori_loop(0, _E, elem_body, 0)
        pltpu.sync_copy(ov, out_hbm.at[pl.ds(base * D, _E * D)])
        return 0

    lax.fori_loop(0, _NCH, chunk_body, 0)


def kernel(x, w1, b1, w2, b2, emb, emb_pad):
    mesh = plsc.VectorSubcoreMesh(core_axis_name="c", subcore_axis_name="s")
    f = pl.kernel(
        _sc_body, mesh=mesh,
        out_type=jax.ShapeDtypeStruct((B * L * D,), jnp.float32),
        scratch_types=[
            pltpu.VMEM((_E,), jnp.float32),
            pltpu.VMEM((_WT,), jnp.float32),
            pltpu.VMEM(((BIN + 1) * _E,), jnp.float32),
            pltpu.VMEM((_E * D,), jnp.float32),
        ],
    )
    w2p = jnp.pad(w2, ((0, 0), (0, 4))).reshape(-1)
    wt = jnp.concatenate([
        jnp.pad(w1.reshape(-1), (0, 4)), jnp.pad(b1, (0, 4)),
        jnp.pad(b2, (0, 4)), w2p, emb.reshape(-1), emb_pad.reshape(-1)])
    return f(jnp.concatenate([wt, x.reshape(-1)])).reshape(B, L, D)


# speedup vs baseline: 5.0540x; 5.0540x over previous
import jax
import jax.numpy as jnp
from jax.experimental import pallas as pl

B, L, D, BIN = 4096, 200, 64, 12
BIN_ALPHA = 1.0
PAD_TOKEN_ID = 0.0

_NB = 16384


def _body(x_ref, w1_ref, b1_ref, w2_ref, b2_ref, emb_ref, pad_ref, o_ref):
    x = x_ref[...].reshape(1, _NB)
    w1c = w1_ref[...].reshape(BIN, 1)
    b1c = b1_ref[...].reshape(BIN, 1)
    b2c = b2_ref[...].reshape(BIN, 1)
    h = x * w1c + b1c
    h = jnp.maximum(h, 0.1 * h)
    h2 = jax.lax.dot_general(w2_ref[...], h, (((0,), (0,)), ((), ())),
                             preferred_element_type=jnp.float32)
    logits = BIN_ALPHA * h + h2 + b2c
    m = jnp.max(logits, axis=0, keepdims=True)
    e = jnp.exp(logits - m)
    w = e * (1.0 / jnp.sum(e, axis=0, keepdims=True))
    pad = (x == PAD_TOKEN_ID)
    w13 = jnp.concatenate([jnp.where(pad, 0.0, w),
                           jnp.where(pad, 1.0, jnp.zeros_like(x))], axis=0)
    emb13 = jnp.concatenate([emb_ref[...], pad_ref[...]], axis=0)
    o_ref[...] = jax.lax.dot_general(w13, emb13, (((0,), (0,)), ((), ())),
                                     preferred_element_type=jnp.float32)


def kernel(x, w1, b1, w2, b2, emb, emb_pad):
    n = B * L
    x_rows = x.reshape(n // _NB, 1, _NB)
    small = pl.BlockSpec(index_map=lambda i: (0, 0))
    out = pl.pallas_call(
        _body,
        grid=(n // _NB,),
        in_specs=[
            pl.BlockSpec((1, 1, _NB), index_map=lambda i: (i, 0, 0)),
            small, small, small, small, small, small,
        ],
        out_specs=pl.BlockSpec((_NB, D), index_map=lambda i: (i, 0)),
        out_shape=jax.ShapeDtypeStruct((n, D), jnp.float32),
    )(x_rows, w1, b1.reshape(1, BIN), w2, b2.reshape(1, BIN), emb, emb_pad)
    return out.reshape(B, L, D)


# speedup vs baseline: 5.1739x; 1.0237x over previous
import jax
import jax.numpy as jnp
from jax.experimental import pallas as pl

B, L, D, BIN = 4096, 200, 64, 12
BIN_ALPHA = 1.0
PAD_TOKEN_ID = 0.0

_NB = 32768


def _body(x_ref, w1_ref, b1_ref, w2_ref, b2_ref, emb_ref, pad_ref, o_ref):
    x = x_ref[...].reshape(1, _NB)
    w1c = w1_ref[...].reshape(BIN, 1)
    b1c = b1_ref[...].reshape(BIN, 1)
    b2c = b2_ref[...].reshape(BIN, 1)
    h = x * w1c + b1c
    h = jnp.maximum(h, 0.1 * h)
    h2 = jax.lax.dot_general(w2_ref[...], h, (((0,), (0,)), ((), ())),
                             preferred_element_type=jnp.float32)
    logits = BIN_ALPHA * h + h2 + b2c
    m = jnp.max(logits, axis=0, keepdims=True)
    e = jnp.exp(logits - m)
    w = e * (1.0 / jnp.sum(e, axis=0, keepdims=True))
    pad = (x == PAD_TOKEN_ID)
    w13 = jnp.concatenate([jnp.where(pad, 0.0, w),
                           jnp.where(pad, 1.0, jnp.zeros_like(x))], axis=0)
    emb13 = jnp.concatenate([emb_ref[...], pad_ref[...]], axis=0)
    o_ref[...] = jax.lax.dot_general(w13, emb13, (((0,), (0,)), ((), ())),
                                     preferred_element_type=jnp.float32)


def kernel(x, w1, b1, w2, b2, emb, emb_pad):
    n = B * L
    x_rows = x.reshape(n // _NB, 1, _NB)
    small = pl.BlockSpec(index_map=lambda i: (0, 0))
    out = pl.pallas_call(
        _body,
        grid=(n // _NB,),
        in_specs=[
            pl.BlockSpec((1, 1, _NB), index_map=lambda i: (i, 0, 0)),
            small, small, small, small, small, small,
        ],
        out_specs=pl.BlockSpec((_NB, D), index_map=lambda i: (i, 0)),
        out_shape=jax.ShapeDtypeStruct((n, D), jnp.float32),
    )(x_rows, w1, b1.reshape(1, BIN), w2, b2.reshape(1, BIN), emb, emb_pad)
    return out.reshape(B, L, D)


# speedup vs baseline: 5.1936x; 1.0038x over previous
import jax
import jax.numpy as jnp
from jax.experimental import pallas as pl

B, L, D, BIN = 4096, 200, 64, 12
BIN_ALPHA = 1.0
PAD_TOKEN_ID = 0.0

_NB = 51200


def _body(x_ref, w1_ref, b1_ref, w2_ref, b2_ref, emb_ref, pad_ref, o_ref):
    x = x_ref[...].reshape(1, _NB)
    w1c = w1_ref[...].reshape(BIN, 1)
    b1c = b1_ref[...].reshape(BIN, 1)
    b2c = b2_ref[...].reshape(BIN, 1)
    h = x * w1c + b1c
    h = jnp.maximum(h, 0.1 * h)
    h2 = jax.lax.dot_general(w2_ref[...], h, (((0,), (0,)), ((), ())),
                             preferred_element_type=jnp.float32)
    logits = BIN_ALPHA * h + h2 + b2c
    m = jnp.max(logits, axis=0, keepdims=True)
    e = jnp.exp(logits - m)
    w = e * (1.0 / jnp.sum(e, axis=0, keepdims=True))
    pad = (x == PAD_TOKEN_ID)
    w13 = jnp.concatenate([jnp.where(pad, 0.0, w),
                           jnp.where(pad, 1.0, jnp.zeros_like(x))], axis=0)
    emb13 = jnp.concatenate([emb_ref[...], pad_ref[...]], axis=0)
    o_ref[...] = jax.lax.dot_general(w13, emb13, (((0,), (0,)), ((), ())),
                                     preferred_element_type=jnp.float32)


def kernel(x, w1, b1, w2, b2, emb, emb_pad):
    n = B * L
    x_rows = x.reshape(n // _NB, 1, _NB)
    small = pl.BlockSpec(index_map=lambda i: (0, 0))
    out = pl.pallas_call(
        _body,
        grid=(n // _NB,),
        in_specs=[
            pl.BlockSpec((1, 1, _NB), index_map=lambda i: (i, 0, 0)),
            small, small, small, small, small, small,
        ],
        out_specs=pl.BlockSpec((_NB, D), index_map=lambda i: (i, 0)),
        out_shape=jax.ShapeDtypeStruct((n, D), jnp.float32),
    )(x_rows, w1, b1.reshape(1, BIN), w2, b2.reshape(1, BIN), emb, emb_pad)
    return out.reshape(B, L, D)
